# Initial kernel scaffold; baseline (speedup 1.0000x reference)
import jax
import jax.numpy as jnp
from jax.experimental import pallas as pl


def kernel(input_ids, word_emb, pos_emb, type_emb, ln_weight, ln_bias):
    raise NotImplementedError("write your pallas kernel here")


# speedup vs baseline: 1.9330x; 1.9330x over previous
import functools

import jax
import jax.numpy as jnp
from jax import lax
from jax.experimental import pallas as pl
from jax.experimental.pallas import tpu as pltpu
from jax.experimental.pallas import tpu_sc as plsc

VOCAB = 100000
HIDDEN = 768
PAD_IDX = 1
LN_EPS = 1e-12
LANES = 16

NC, NS = 2, 16
NW = NC * NS


def _sc_gather_add_body(word_hbm, pos_hbm, ids_hbm, pid_hbm, out_hbm,
                        idx_w, idx_p, buf_w, buf_p, sem_w, sem_p,
                        *, tokens_per_worker, chunk):
    wid = lax.axis_index("s") * NC + lax.axis_index("c")
    base = wid * tokens_per_worker
    nchunks = tokens_per_worker // chunk

    def chunk_body(ci, _):
        off = base + ci * chunk
        pltpu.sync_copy(ids_hbm.at[pl.ds(off, chunk)], idx_w)
        pltpu.sync_copy(pid_hbm.at[pl.ds(off, chunk)], idx_p)
        cw = pltpu.async_copy(word_hbm.at[idx_w], buf_w, sem_w)
        cp = pltpu.async_copy(pos_hbm.at[idx_p], buf_p, sem_p)
        cw.wait()
        cp.wait()

        def add_row(r, _):
            for j in range(HIDDEN // LANES):
                sl = pl.ds(j * LANES, LANES)
                buf_w[r, sl] = buf_w[r, sl] + buf_p[r, sl]
            return 0

        lax.fori_loop(0, chunk, add_row, 0)
        pltpu.sync_copy(buf_w, out_hbm.at[pl.ds(off, chunk)])
        return 0

    lax.fori_loop(0, nchunks, chunk_body, 0)


def _make_sc_gather_add(n_tokens, chunk):
    tokens_per_worker = n_tokens // NW
    body = functools.partial(_sc_gather_add_body,
                             tokens_per_worker=tokens_per_worker, chunk=chunk)
    return pl.kernel(
        body,
        out_type=jax.ShapeDtypeStruct((n_tokens, HIDDEN), jnp.float32),
        mesh=plsc.VectorSubcoreMesh(core_axis_name="c", subcore_axis_name="s",
                                    num_cores=NC, num_subcores=NS),
        scratch_types=[
            pltpu.VMEM((chunk,), jnp.int32),
            pltpu.VMEM((chunk,), jnp.int32),
            pltpu.VMEM((chunk, HIDDEN), jnp.float32),
            pltpu.VMEM((chunk, HIDDEN), jnp.float32),
            pltpu.SemaphoreType.DMA,
            pltpu.SemaphoreType.DMA,
        ],
    )


def _ln_kernel(x_ref, tvec_ref, w_ref, b_ref, o_ref):
    e = x_ref[...] + tvec_ref[...]
    mean = jnp.mean(e, axis=-1, keepdims=True)
    c = e - mean
    var = jnp.mean(c * c, axis=-1, keepdims=True)
    o_ref[...] = (c * lax.rsqrt(var + LN_EPS)) * w_ref[...] + b_ref[...]


def _layernorm(summed, tvec, w, b, block_rows):
    n = summed.shape[0]
    return pl.pallas_call(
        _ln_kernel,
        grid=(n // block_rows,),
        in_specs=[
            pl.BlockSpec((block_rows, HIDDEN), lambda i: (i, 0)),
            pl.BlockSpec((1, HIDDEN), lambda i: (0, 0)),
            pl.BlockSpec((1, HIDDEN), lambda i: (0, 0)),
            pl.BlockSpec((1, HIDDEN), lambda i: (0, 0)),
        ],
        out_specs=pl.BlockSpec((block_rows, HIDDEN), lambda i: (i, 0)),
        out_shape=jax.ShapeDtypeStruct((n, HIDDEN), jnp.float32),
    )(summed, tvec, w, b)


def kernel(input_ids, word_emb, pos_emb, type_emb, ln_weight, ln_bias):
    B, S = input_ids.shape
    n = B * S
    ids = input_ids.reshape(-1).astype(jnp.int32)
    mask = (input_ids != PAD_IDX).astype(jnp.int32)
    pos = (jnp.cumsum(mask, axis=1) * mask + PAD_IDX).astype(jnp.int32)
    pos = pos.reshape(-1)

    gather = _make_sc_gather_add(n, chunk=64)
    summed = gather(word_emb, pos_emb, ids, pos)

    out = _layernorm(summed,
                     type_emb[0].reshape(1, HIDDEN),
                     ln_weight.reshape(1, HIDDEN),
                     ln_bias.reshape(1, HIDDEN),
                     block_rows=512)
    return out.reshape(B, S, HIDDEN)


# speedup vs baseline: 2.5805x; 1.3349x over previous
import functools

import jax
import jax.numpy as jnp
from jax import lax
from jax.experimental import pallas as pl
from jax.experimental.pallas import tpu as pltpu
from jax.experimental.pallas import tpu_sc as plsc

VOCAB = 100000
HIDDEN = 768
PAD_IDX = 1
LN_EPS = 1e-12
LANES = 16

NC, NS = 2, 16
NW = NC * NS


def _sc_gather_add_body(word_hbm, pos_hbm, ids_hbm, pid_hbm, out_hbm,
                        ids_v, pid_v, bw0, bw1, bp0, bp1,
                        sw0, sw1, sp0, sp1, so0, so1,
                        *, tokens_per_worker, chunk):
    wid = lax.axis_index("s") * NC + lax.axis_index("c")
    base = wid * tokens_per_worker
    nchunks = tokens_per_worker // chunk
    npairs = nchunks // 2
    bw = (bw0, bw1)
    bp = (bp0, bp1)
    sw = (sw0, sw1)
    sp = (sp0, sp1)
    so = (so0, so1)

    pltpu.sync_copy(ids_hbm.at[pl.ds(base, tokens_per_worker)], ids_v)
    pltpu.sync_copy(pid_hbm.at[pl.ds(base, tokens_per_worker)], pid_v)

    def start_gathers(ci, b):
        sl = pl.ds(ci * chunk, chunk)
        pltpu.async_copy(word_hbm.at[ids_v.at[sl]], bw[b], sw[b])
        pltpu.async_copy(pos_hbm.at[pid_v.at[sl]], bp[b], sp[b])

    start_gathers(0, 0)
    start_gathers(1, 1)

    def pair_body_wrap(k, carry):
        for b in (0, 1):
            ci = 2 * k + b
            pltpu.make_async_copy(word_hbm.at[ids_v.at[pl.ds(0, chunk)]],
                                  bw[b], sw[b]).wait()
            pltpu.make_async_copy(pos_hbm.at[pid_v.at[pl.ds(0, chunk)]],
                                  bp[b], sp[b]).wait()

            def add_row(r, _):
                for j in range(HIDDEN // LANES):
                    sl = pl.ds(j * LANES, LANES)
                    plsc.addupdate(bw[b].at[r, sl], bp[b][r, sl])
                return 0

            lax.fori_loop(0, chunk, add_row, 0)
            out_sl = pl.ds(base + ci * chunk, chunk)
            pltpu.async_copy(bw[b], out_hbm.at[out_sl], so[b])

            @pl.when(ci + 2 < nchunks)
            def _():
                pltpu.make_async_copy(bw[b], out_hbm.at[out_sl], so[b]).wait()
                start_gathers(ci + 2, b)
        return carry

    lax.fori_loop(0, npairs, pair_body_wrap, 0)

    last_sl = pl.ds(base + (nchunks - 2) * chunk, chunk)
    pltpu.make_async_copy(bw[0], out_hbm.at[last_sl], so[0]).wait()
    pltpu.make_async_copy(bw[1], out_hbm.at[last_sl], so[1]).wait()


def _make_sc_gather_add(n_tokens, chunk):
    tokens_per_worker = n_tokens // NW
    body = functools.partial(_sc_gather_add_body,
                             tokens_per_worker=tokens_per_worker, chunk=chunk)
    return pl.kernel(
        body,
        out_type=jax.ShapeDtypeStruct((n_tokens, HIDDEN), jnp.float32),
        mesh=plsc.VectorSubcoreMesh(core_axis_name="c", subcore_axis_name="s",
                                    num_cores=NC, num_subcores=NS),
        scratch_types=[
            pltpu.VMEM((tokens_per_worker,), jnp.int32),
            pltpu.VMEM((tokens_per_worker,), jnp.int32),
            pltpu.VMEM((chunk, HIDDEN), jnp.float32),
            pltpu.VMEM((chunk, HIDDEN), jnp.float32),
            pltpu.VMEM((chunk, HIDDEN), jnp.float32),
            pltpu.VMEM((chunk, HIDDEN), jnp.float32),
            pltpu.SemaphoreType.DMA,
            pltpu.SemaphoreType.DMA,
            pltpu.SemaphoreType.DMA,
            pltpu.SemaphoreType.DMA,
            pltpu.SemaphoreType.DMA,
            pltpu.SemaphoreType.DMA,
        ],
    )


def _ln_kernel(x_ref, tvec_ref, w_ref, b_ref, o_ref):
    e = x_ref[...] + tvec_ref[...]
    mean = jnp.mean(e, axis=-1, keepdims=True)
    c = e - mean
    var = jnp.mean(c * c, axis=-1, keepdims=True)
    o_ref[...] = (c * lax.rsqrt(var + LN_EPS)) * w_ref[...] + b_ref[...]


def _layernorm(summed, tvec, w, b, block_rows):
    n = summed.shape[0]
    return pl.pallas_call(
        _ln_kernel,
        grid=(n // block_rows,),
        in_specs=[
            pl.BlockSpec((block_rows, HIDDEN), lambda i: (i, 0)),
            pl.BlockSpec((1, HIDDEN), lambda i: (0, 0)),
            pl.BlockSpec((1, HIDDEN), lambda i: (0, 0)),
            pl.BlockSpec((1, HIDDEN), lambda i: (0, 0)),
        ],
        out_specs=pl.BlockSpec((block_rows, HIDDEN), lambda i: (i, 0)),
        out_shape=jax.ShapeDtypeStruct((n, HIDDEN), jnp.float32),
    )(summed, tvec, w, b)


def kernel(input_ids, word_emb, pos_emb, type_emb, ln_weight, ln_bias):
    B, S = input_ids.shape
    n = B * S
    ids = input_ids.reshape(-1).astype(jnp.int32)
    mask = (input_ids != PAD_IDX).astype(jnp.int32)
    pos = (jnp.cumsum(mask, axis=1) * mask + PAD_IDX).astype(jnp.int32)
    pos = pos.reshape(-1)

    gather = _make_sc_gather_add(n, chunk=32)
    summed = gather(word_emb, pos_emb, ids, pos)

    out = _layernorm(summed,
                     type_emb[0].reshape(1, HIDDEN),
                     ln_weight.reshape(1, HIDDEN),
                     ln_bias.reshape(1, HIDDEN),
                     block_rows=512)
    return out.reshape(B, S, HIDDEN)


# speedup vs baseline: 2.7967x; 1.0838x over previous
import functools

import jax
import jax.numpy as jnp
from jax import lax
from jax.experimental import pallas as pl
from jax.experimental.pallas import tpu as pltpu
from jax.experimental.pallas import tpu_sc as plsc

VOCAB = 100000
HIDDEN = 768
PAD_IDX = 1
LN_EPS = 1e-12
LANES = 16

NC, NS = 2, 16
NW = NC * NS


def _sc_gather_add_body(word_hbm, pos_hbm, ids_hbm, pid_hbm, out_hbm,
                        ids_v, pid_v, bw0, bw1, bp0, bp1,
                        sw0, sw1, sp0, sp1, so0, so1,
                        *, tokens_per_worker, chunk):
    wid = lax.axis_index("s") * NC + lax.axis_index("c")
    base = wid * tokens_per_worker
    nchunks = tokens_per_worker // chunk
    npairs = nchunks // 2
    bw = (bw0, bw1)
    bp = (bp0, bp1)
    sw = (sw0, sw1)
    sp = (sp0, sp1)
    so = (so0, so1)

    pltpu.sync_copy(ids_hbm.at[pl.ds(base, tokens_per_worker)], ids_v)
    pltpu.sync_copy(pid_hbm.at[pl.ds(base, tokens_per_worker)], pid_v)

    def start_gathers(ci, b):
        sl = pl.ds(ci * chunk, chunk)
        pltpu.async_copy(word_hbm.at[ids_v.at[sl]], bw[b], sw[b])
        pltpu.async_copy(pos_hbm.at[pid_v.at[sl]], bp[b], sp[b])

    start_gathers(0, 0)
    start_gathers(1, 1)

    def pair_body_wrap(k, carry):
        for b in (0, 1):
            ci = 2 * k + b
            pltpu.make_async_copy(word_hbm.at[ids_v.at[pl.ds(0, chunk)]],
                                  bw[b], sw[b]).wait()
            pltpu.make_async_copy(pos_hbm.at[pid_v.at[pl.ds(0, chunk)]],
                                  bp[b], sp[b]).wait()

            def add_row(r, _):
                for j in range(HIDDEN // LANES):
                    sl = pl.ds(j * LANES, LANES)
                    plsc.addupdate(bw[b].at[r, sl], bp[b][r, sl])
                return 0

            lax.fori_loop(0, chunk, add_row, 0)
            out_sl = pl.ds(base + ci * chunk, chunk)
            pltpu.async_copy(bw[b], out_hbm.at[out_sl], so[b])

            @pl.when(ci + 2 < nchunks)
            def _():
                pltpu.make_async_copy(bw[b], out_hbm.at[out_sl], so[b]).wait()
                start_gathers(ci + 2, b)
        return carry

    lax.fori_loop(0, npairs, pair_body_wrap, 0)

    last_sl = pl.ds(base + (nchunks - 2) * chunk, chunk)
    pltpu.make_async_copy(bw[0], out_hbm.at[last_sl], so[0]).wait()
    pltpu.make_async_copy(bw[1], out_hbm.at[last_sl], so[1]).wait()


def _make_sc_gather_add(n_tokens, chunk):
    tokens_per_worker = n_tokens // NW
    body = functools.partial(_sc_gather_add_body,
                             tokens_per_worker=tokens_per_worker, chunk=chunk)
    return pl.kernel(
        body,
        out_type=jax.ShapeDtypeStruct((n_tokens, HIDDEN), jnp.float32),
        mesh=plsc.VectorSubcoreMesh(core_axis_name="c", subcore_axis_name="s",
                                    num_cores=NC, num_subcores=NS),
        scratch_types=[
            pltpu.VMEM((tokens_per_worker,), jnp.int32),
            pltpu.VMEM((tokens_per_worker,), jnp.int32),
            pltpu.VMEM((chunk, HIDDEN), jnp.float32),
            pltpu.VMEM((chunk, HIDDEN), jnp.float32),
            pltpu.VMEM((chunk, HIDDEN), jnp.float32),
            pltpu.VMEM((chunk, HIDDEN), jnp.float32),
            pltpu.SemaphoreType.DMA,
            pltpu.SemaphoreType.DMA,
            pltpu.SemaphoreType.DMA,
            pltpu.SemaphoreType.DMA,
            pltpu.SemaphoreType.DMA,
            pltpu.SemaphoreType.DMA,
        ],
    )


def _ln_kernel(x_ref, tvec_ref, w_ref, b_ref, o_ref):
    e = x_ref[...] + tvec_ref[...]
    mean = jnp.mean(e, axis=-1, keepdims=True)
    c = e - mean
    var = jnp.mean(c * c, axis=-1, keepdims=True)
    o_ref[...] = (c * lax.rsqrt(var + LN_EPS)) * w_ref[...] + b_ref[...]


def _layernorm(summed, tvec, w, b, block_rows):
    n = summed.shape[0]
    return pl.pallas_call(
        _ln_kernel,
        grid=(n // block_rows,),
        in_specs=[
            pl.BlockSpec((block_rows, HIDDEN), lambda i: (i, 0)),
            pl.BlockSpec((1, HIDDEN), lambda i: (0, 0)),
            pl.BlockSpec((1, HIDDEN), lambda i: (0, 0)),
            pl.BlockSpec((1, HIDDEN), lambda i: (0, 0)),
        ],
        out_specs=pl.BlockSpec((block_rows, HIDDEN), lambda i: (i, 0)),
        out_shape=jax.ShapeDtypeStruct((n, HIDDEN), jnp.float32),
    )(summed, tvec, w, b)


def kernel(input_ids, word_emb, pos_emb, type_emb, ln_weight, ln_bias):
    B, S = input_ids.shape
    n = B * S
    ids = input_ids.reshape(-1).astype(jnp.int32)
    mask = (input_ids != PAD_IDX).astype(jnp.int32)
    pos = (jnp.cumsum(mask, axis=1) * mask + PAD_IDX).astype(jnp.int32)
    pos = pos.reshape(-1)

    gather = _make_sc_gather_add(n, chunk=32)
    summed = gather(word_emb, pos_emb, ids, pos)

    out = _layernorm(summed,
                     type_emb[0].reshape(1, HIDDEN),
                     ln_weight.reshape(1, HIDDEN),
                     ln_bias.reshape(1, HIDDEN),
                     block_rows=1024)
    return out.reshape(B, S, HIDDEN)
